# baseline (speedup 1.0000x reference)
import functools

import jax
import jax.numpy as jnp
from jax import lax
from jax.experimental import pallas as pl
from jax.experimental.pallas import tpu as pltpu
from jax.experimental.pallas import tpu_sc as plsc

NC = 2
NS = 16
L = 16
NW = NC * NS

D = 64
B = 16384
BPW = B // NW
GROUPS = BPW // L

_mesh = plsc.VectorSubcoreMesh(core_axis_name="c", subcore_axis_name="s")


@functools.partial(
    pl.kernel,
    out_type=jax.ShapeDtypeStruct((B,), jnp.float32),
    mesh=_mesh,
    scratch_types=[
        pltpu.VMEM((BPW,), jnp.int32),
        pltpu.VMEM((BPW,), jnp.int32),
        pltpu.VMEM((BPW, D), jnp.float32),
        pltpu.VMEM((BPW, D), jnp.float32),
        pltpu.VMEM((BPW,), jnp.float32),
        pltpu.SemaphoreType.DMA,
    ],
    compiler_params=pltpu.CompilerParams(needs_layout_passes=False,
                                         use_tc_tiling_on_sc=False),
)
def _two_tower_sc(u_idx_hbm, i_idx_hbm, u_tab_hbm, i_tab_hbm, out_hbm,
                  u_idx_v, i_idx_v, u_rows, i_rows, out_v, sem):
    wid = lax.axis_index("s") * NC + lax.axis_index("c")
    base = wid * BPW

    pltpu.sync_copy(u_idx_hbm.at[pl.ds(base, BPW)], u_idx_v)
    pltpu.sync_copy(i_idx_hbm.at[pl.ds(base, BPW)], i_idx_v)
    cu = pltpu.async_copy(u_tab_hbm.at[u_idx_v], u_rows, sem)
    ci = pltpu.async_copy(i_tab_hbm.at[i_idx_v], i_rows, sem)
    cu.wait()
    ci.wait()

    def group_body(g, carry):
        rows = g * L + lax.iota(jnp.int32, L)
        acc = jnp.zeros((L,), jnp.float32)
        for d in range(D):
            col = jnp.full((L,), d, jnp.int32)
            acc = acc + (plsc.load_gather(u_rows, [rows, col])
                         * plsc.load_gather(i_rows, [rows, col]))
        out_v[pl.ds(g * L, L)] = acc
        return carry

    lax.fori_loop(0, GROUPS, group_body, 0)
    pltpu.sync_copy(out_v, out_hbm.at[pl.ds(base, BPW)])


def kernel(user_input, item_input, user_table, item_table):
    out = _two_tower_sc(user_input.astype(jnp.int32),
                        item_input.astype(jnp.int32),
                        user_table, item_table)
    return out.reshape(B, 1)
